# baseline (speedup 1.0000x reference)
import functools

import jax
import jax.numpy as jnp
from jax import lax
from jax.experimental import pallas as pl
from jax.experimental.pallas import tpu as pltpu
from jax.experimental.pallas import tpu_sc as plsc

N = 10000
E = 320000
DIM = 128
NG = 64
NH = 2
HD = DIM // NH
NS = 25
R = 1000
GRID = N // R
SCALE = 1.0 / (DIM ** 0.5)
F32 = jnp.float32


def _relu(v):
    return jnp.maximum(v, 0.0)


def _dot(a, b):
    return lax.dot_general(a, b, (((1,), (0,)), ((), ())),
                           preferred_element_type=F32)


def _dot3(a, w):
    return lax.dot_general(a, w, (((2,), (0,)), ((), ())),
                           preferred_element_type=F32)


_row = pl.BlockSpec((R, DIM), lambda i: (i, 0))
_row2 = pl.BlockSpec((R, 2 * DIM), lambda i: (i, 0))
_seg = pl.BlockSpec((2, R, DIM), lambda i: (0, i, 0))
_col = pl.BlockSpec((R, 1), lambda i: (i, 0))
_mat = pl.BlockSpec((DIM, DIM), lambda i: (0, 0))
_bia = pl.BlockSpec((1, DIM), lambda i: (0, 0))


def _mlp3_body(x, w1, b1, w2, b2, w3, b3, o):
    h = _relu(_dot(x[...], w1[...]) + b1[...])
    h = _relu(_dot(h, w2[...]) + b2[...])
    o[...] = _relu(_dot(h, w3[...]) + b3[...])


def _cntred_body(cnt, o):
    o[...] = cnt[0, :, 0:1] + cnt[1, :, 0:1]


def _conv_lin_body(s, cnt, h, wl, bl, wr, w4, b4, o):
    agg = (s[0] + s[1]) / jnp.maximum(cnt[...], 1.0)
    hh = _relu(_dot(agg, wl[...]) + bl[...] + _dot(h[...], wr[...]))
    o[...] = _relu(_dot(hh, w4[...]) + b4[...])


def _conv_gcnprep_body(s, cnt, h, wl, bl, wr, w6, b6, wg, bg, wk, wv,
                       hk_o, hv_o, dinv_o):
    agg = (s[0] + s[1]) / jnp.maximum(cnt[...], 1.0)
    hh = _relu(_dot(agg, wl[...]) + bl[...] + _dot(h[...], wr[...]))
    h6 = _relu(_dot(hh, w6[...]) + b6[...])
    xg = _dot(h6, wg[...]) + bg[...]
    dinv = lax.rsqrt(cnt[...] + 1.0)
    hk_o[...] = _dot(xg, wk[...]) * dinv
    hv_o[...] = _dot(xg, wv[...]) * dinv
    dinv_o[...] = dinv


def _pool1_body(skv, hk, hv, dinv, batch, s1, wq, bq, bk, bv, den_o, num_o):
    i = pl.program_id(0)

    @pl.when(i == 0)
    def _():
        den_o[...] = jnp.zeros_like(den_o)
        num_o[...] = jnp.zeros_like(num_o)

    di = dinv[...]
    K = di * (skv[0] + hk[...]) + bk[...]
    V = di * (skv[1] + hv[...]) + bv[...]
    Qp = _dot(s1[...], wq[...]) + bq[...]
    onehot = (batch[...] == lax.broadcasted_iota(jnp.int32, (R, NG), 1)
              ).astype(F32)
    for h in range(NH):
        sl = slice(h * HD, (h + 1) * HD)
        Kh, Vh, Qh = K[:, sl], V[:, sl], Qp[:, sl]
        S = lax.dot_general(Kh, Qh, (((1,), (1,)), ((), ())),
                            preferred_element_type=F32) * SCALE
        w = jnp.exp(S)
        den_o[:, h * NS:(h + 1) * NS] += lax.dot_general(
            onehot, w, (((0,), (0,)), ((), ())), preferred_element_type=F32)
        wb = (w[:, :, None] * onehot[:, None, :]).reshape(R, NS * NG)
        num_o[h] += lax.dot_general(
            wb, Vh, (((0,), (0,)), ((), ())), preferred_element_type=F32)


def _pooltail_body(num, den, s1, s3,
                   wq1, bq1, wo1, bo1,
                   wq2, bq2, wk2, bk2, wv2, bv2, wo2, bo2,
                   wq3, bq3, wk3, bk3, wv3, bv3, wo3, bo3,
                   wg2, bg2, out):
    Qp1 = _dot(s1[...], wq1[...]) + bq1[...]
    d = den[...]
    rows = []
    for q in range(NS):
        d0 = d[:, q:q + 1]
        d1 = d[:, NS + q:NS + q + 1]
        n0 = num[0, q * NG:(q + 1) * NG, :]
        n1 = num[1, q * NG:(q + 1) * NG, :]
        v0 = jnp.where(d0 > 0, n0 / jnp.maximum(d0, 1e-30), 0.0)
        v1 = jnp.where(d1 > 0, n1 / jnp.maximum(d1, 1e-30), 0.0)
        o = jnp.concatenate([v0, v1], axis=1) + Qp1[q:q + 1, :]
        rows.append(o + _relu(_dot(o, wo1[...]) + bo1[...]))
    bx = jnp.stack(rows, axis=1)

    def lin3(a, w, b):
        return _dot3(a, w[...]) + b[...]

    def sab(bx2, Qf, wk, bk_, wv, bv_, wo, bo_):
        K2 = lin3(bx2, wk, bk_)
        V2 = lin3(bx2, wv, bv_)
        outs = []
        for h in range(NH):
            sl = slice(h * HD, (h + 1) * HD)
            sc = lax.dot_general(Qf[..., sl], K2[..., sl],
                                 (((2,), (2,)), ((0,), (0,))),
                                 preferred_element_type=F32) * SCALE
            m = jnp.max(sc, axis=-1, keepdims=True)
            e = jnp.exp(sc - m)
            A = e / jnp.sum(e, axis=-1, keepdims=True)
            outs.append(Qf[..., sl] + lax.dot_general(
                A, V2[..., sl], (((2,), (1,)), ((0,), (0,))),
                preferred_element_type=F32))
        o2 = jnp.concatenate(outs, axis=-1)
        return o2 + _relu(lin3(o2, wo, bo_))

    bx = sab(bx, lin3(bx, wq2, bq2), wk2, bk2, wv2, bv2, wo2, bo2)

    Qp3 = _dot(s3[...], wq3[...]) + bq3[...]
    K3 = lin3(bx, wk3, bk3)
    V3 = lin3(bx, wv3, bv3)
    outs = []
    for h in range(NH):
        sl = slice(h * HD, (h + 1) * HD)
        sc = jnp.sum(K3[..., sl] * Qp3[0:1, None, sl], axis=-1) * SCALE
        m = jnp.max(sc, axis=-1, keepdims=True)
        e = jnp.exp(sc - m)
        A = e / jnp.sum(e, axis=-1, keepdims=True)
        outs.append(Qp3[0:1, sl] + jnp.sum(A[..., None] * V3[..., sl], axis=1))
    o3 = jnp.concatenate(outs, axis=-1)
    bx3 = o3 + _relu(_dot(o3, wo3[...]) + bo3[...])
    out[...] = _dot(bx3, wg2[...]) + bg2[...]


SC_NC = 2
SC_NS = 16
NW = SC_NC * SC_NS
EW = E // NW
CH = 128
E_PAD = 323584
EWP = E_PAD // NW
NCH = EWP // CH
N_PAD = 10240
ZR = N_PAD // SC_NS
WB = 80
NWB = ZR // WB
_MESH = plsc.VectorSubcoreMesh(core_axis_name="c", subcore_axis_name="s",
                               num_cores=SC_NC, num_subcores=SC_NS)


def _zero_acc(zrow, buf, acc, s):
    pltpu.sync_copy(zrow, buf)
    for j in range(NWB):
        pltpu.sync_copy(buf, acc.at[pl.ds(s * ZR + j * WB, WB)])


def _writeback(acc, buf, out, c, s):
    for j in range(NWB):
        off = s * ZR + j * WB
        pltpu.sync_copy(acc.at[pl.ds(off, WB)], buf)
        pltpu.sync_copy(buf, out.at[pl.ds(c * N_PAD + off, WB)])


def _make_segsum():
    scratch = [
        pltpu.VMEM((CH,), jnp.int32),
        pltpu.VMEM((CH,), jnp.int32),
        pltpu.VMEM((CH,), jnp.int32),
        pltpu.VMEM((CH,), jnp.int32),
        pltpu.VMEM((CH, DIM), F32),
        pltpu.VMEM((CH, DIM), F32),
        pltpu.VMEM_SHARED((N_PAD, DIM), F32),
        pltpu.SemaphoreType.DMA,
        pltpu.SemaphoreType.DMA,
        pltpu.SemaphoreType.DMA,
        pltpu.SemaphoreType.DMA,
    ]

    def body(tab, srcr, dstr, zrow, out,
             sa_i, da_i, sb_i, db_i, ra, rb, acc, ga, gb, ia, ib):
        c = lax.axis_index("c")
        s = lax.axis_index("s")
        wid = s * SC_NC + c
        base = wid * EWP
        wbuf = ra.at[pl.ds(0, WB)]
        _zero_acc(zrow, wbuf, acc, s)
        plsc.subcore_barrier()

        def load_idx(k, si, di, sem):
            pltpu.async_copy(srcr.at[pl.ds(base + k * CH, CH)], si, sem)
            pltpu.async_copy(dstr.at[pl.ds(base + k * CH, CH)], di, sem)

        def wait_idx(si, di, sem):
            pltpu.make_async_copy(srcr.at[pl.ds(0, CH)], si, sem).wait()
            pltpu.make_async_copy(dstr.at[pl.ds(0, CH)], di, sem).wait()

        load_idx(0, sa_i, da_i, ia)
        wait_idx(sa_i, da_i, ia)
        pltpu.async_copy(tab.at[sa_i], ra, ga)
        load_idx(1, sb_i, db_i, ib)

        def step(i, carry):
            pltpu.make_async_copy(tab.at[sa_i], ra, ga).wait()
            wait_idx(sb_i, db_i, ib)
            pltpu.async_copy(tab.at[sb_i], rb, gb)
            pltpu.sync_copy(ra, acc.at[da_i], add=True)

            @pl.when(2 * i + 2 < NCH)
            def _():
                load_idx(2 * i + 2, sa_i, da_i, ia)

            pltpu.make_async_copy(tab.at[sb_i], rb, gb).wait()
            pltpu.sync_copy(rb, acc.at[db_i], add=True)

            @pl.when(2 * i + 2 < NCH)
            def _():
                wait_idx(sa_i, da_i, ia)
                pltpu.async_copy(tab.at[sa_i], ra, ga)

            @pl.when(2 * i + 3 < NCH)
            def _():
                load_idx(2 * i + 3, sb_i, db_i, ib)

            return carry

        lax.fori_loop(0, NCH // 2, step, 0)
        if NCH % 2:
            pltpu.make_async_copy(tab.at[sa_i], ra, ga).wait()
            pltpu.sync_copy(ra, acc.at[da_i], add=True)
        plsc.subcore_barrier()
        _writeback(acc, ra.at[pl.ds(0, WB)], out, c, s)

    return pl.kernel(body,
                     out_type=jax.ShapeDtypeStruct((2 * N_PAD, DIM), F32),
                     mesh=_MESH, scratch_types=scratch)


def _make_segsum_kv():
    EW2 = E_PAD // SC_NS
    NCH2 = EW2 // CH
    scratch = [
        pltpu.VMEM((CH,), jnp.int32),
        pltpu.VMEM((CH,), jnp.int32),
        pltpu.VMEM((CH,), jnp.int32),
        pltpu.VMEM((CH,), jnp.int32),
        pltpu.VMEM((CH, DIM), F32),
        pltpu.VMEM((CH, DIM), F32),
        pltpu.VMEM_SHARED((N_PAD, DIM), F32),
        pltpu.SemaphoreType.DMA,
        pltpu.SemaphoreType.DMA,
        pltpu.SemaphoreType.DMA,
        pltpu.SemaphoreType.DMA,
    ]

    def body(tabk, tabv, srcr, dstr, zrow, out,
             sa_i, da_i, sb_i, db_i, ra, rb, acc, ga, gb, ia, ib):
        c = lax.axis_index("c")
        s = lax.axis_index("s")
        base = s * EW2
        wbuf = ra.at[pl.ds(0, WB)]
        _zero_acc(zrow, wbuf, acc, s)
        plsc.subcore_barrier()

        def load_idx(k, si, di, sem):
            pltpu.async_copy(srcr.at[pl.ds(base + k * CH, CH)], si, sem)
            pltpu.async_copy(dstr.at[pl.ds(base + k * CH, CH)], di, sem)

        def wait_idx(si, di, sem):
            pltpu.make_async_copy(srcr.at[pl.ds(0, CH)], si, sem).wait()
            pltpu.make_async_copy(dstr.at[pl.ds(0, CH)], di, sem).wait()

        def pipeline(tab):
            load_idx(0, sa_i, da_i, ia)
            wait_idx(sa_i, da_i, ia)
            pltpu.async_copy(tab.at[sa_i], ra, ga)
            load_idx(1, sb_i, db_i, ib)

            def step(i, carry):
                pltpu.make_async_copy(tab.at[sa_i], ra, ga).wait()
                wait_idx(sb_i, db_i, ib)
                pltpu.async_copy(tab.at[sb_i], rb, gb)
                pltpu.sync_copy(ra, acc.at[da_i], add=True)

                @pl.when(2 * i + 2 < NCH2)
                def _():
                    load_idx(2 * i + 2, sa_i, da_i, ia)

                pltpu.make_async_copy(tab.at[sb_i], rb, gb).wait()
                pltpu.sync_copy(rb, acc.at[db_i], add=True)

                @pl.when(2 * i + 2 < NCH2)
                def _():
                    wait_idx(sa_i, da_i, ia)
                    pltpu.async_copy(tab.at[sa_i], ra, ga)

                @pl.when(2 * i + 3 < NCH2)
                def _():
                    load_idx(2 * i + 3, sb_i, db_i, ib)

                return carry

            lax.fori_loop(0, NCH2 // 2, step, 0)

        @pl.when(c == 0)
        def _():
            pipeline(tabk)

        @pl.when(c == 1)
        def _():
            pipeline(tabv)

        plsc.subcore_barrier()
        _writeback(acc, ra.at[pl.ds(0, WB)], out, c, s)

    return pl.kernel(body,
                     out_type=jax.ShapeDtypeStruct((2 * N_PAD, DIM), F32),
                     mesh=_MESH, scratch_types=scratch)


def _segsum_kv(tabk, tabv, src, dst):
    zrow = jnp.zeros((WB, DIM), F32)
    srcp, dstp = _pad_edges(src, dst)
    s = _make_segsum_kv()(tabk, tabv, srcp, dstp, zrow)
    return s.reshape(2, N_PAD, DIM)


def _make_counts():
    scratch = [
        pltpu.VMEM((NCH, CH), jnp.int32),
        pltpu.VMEM((CH, DIM), F32),
        pltpu.VMEM_SHARED((N_PAD, DIM), F32),
    ]

    def body(dst3, zrow, ones, out, didx, ob, acc):
        c = lax.axis_index("c")
        s = lax.axis_index("s")
        wid = s * SC_NC + c
        wbuf = ob.at[pl.ds(0, WB)]
        _zero_acc(zrow, wbuf, acc, s)
        pltpu.sync_copy(dst3.at[wid], didx)
        pltpu.sync_copy(ones, ob)
        plsc.subcore_barrier()

        def step(i, carry):
            pltpu.sync_copy(ob, acc.at[didx.at[i]], add=True)
            return carry

        lax.fori_loop(0, NCH, step, 0)
        plsc.subcore_barrier()
        _writeback(acc, wbuf, out, c, s)

    return pl.kernel(body,
                     out_type=jax.ShapeDtypeStruct((2 * N_PAD, DIM), F32),
                     mesh=_MESH, scratch_types=scratch)


def _pad_edges(src, dst):
    pad = E_PAD - E
    sinks = jnp.full((pad,), N_PAD - 8, jnp.int32)
    return (jnp.concatenate([src, jnp.zeros((pad,), jnp.int32)]),
            jnp.concatenate([dst, sinks]))


def _segsum(tab, src, dst):
    zrow = jnp.zeros((WB, DIM), F32)
    srcp, dstp = _pad_edges(src, dst)
    s = _make_segsum()(tab, srcp, dstp, zrow)
    return s.reshape(2, N_PAD, DIM)


def _counts(dst):
    zrow = jnp.zeros((WB, DIM), F32)
    ones = jnp.ones((CH, DIM), F32)
    _, dstp = _pad_edges(jnp.zeros((E,), jnp.int32), dst)
    dst3 = dstp.reshape(NW, NCH, CH)
    c = _make_counts()(dst3, zrow, ones)
    return c.reshape(2, N_PAD, DIM)


def kernel(x, params, edge_index, batch):
    p = params
    src, dst = edge_index[0], edge_index[1]

    def wt(nm):
        return p[nm + '_W'].T

    def wtk(key):
        return p[key].T

    def b2(nm):
        return p[nm + '_b'].reshape(1, DIM)

    def b2k(key):
        return p[key].reshape(1, DIM)

    batch2 = batch.reshape(N, 1)
    cnt2 = _counts(dst)
    cnt = pl.pallas_call(
        _cntred_body, grid=(GRID,),
        in_specs=[_seg],
        out_specs=_col,
        out_shape=jax.ShapeDtypeStruct((N, 1), F32),
    )(cnt2)
    _cnt = _col

    h3 = pl.pallas_call(
        _mlp3_body, grid=(GRID,),
        in_specs=[_row, _mat, _bia, _mat, _bia, _mat, _bia],
        out_specs=_row,
        out_shape=jax.ShapeDtypeStruct((N, DIM), F32),
    )(x, wt('lin1'), b2('lin1'), wt('lin2'), b2('lin2'), wt('lin3'), b2('lin3'))

    h = h3
    for conv, nxt in (('conv1', 'lin4'), ('conv2', 'lin5')):
        s = _segsum(h, src, dst)
        h = pl.pallas_call(
            _conv_lin_body, grid=(GRID,),
            in_specs=[_seg, _cnt, _row, _mat, _bia, _mat, _mat, _bia],
            out_specs=_row,
            out_shape=jax.ShapeDtypeStruct((N, DIM), F32),
        )(s, cnt, h, wtk(conv + '_Wl'), b2k(conv + '_bl'), wtk(conv + '_Wr'),
          wt(nxt), b2(nxt))

    s = _segsum(h, src, dst)
    hk_s, hv_s, dinv = pl.pallas_call(
        _conv_gcnprep_body, grid=(GRID,),
        in_specs=[_seg, _cnt, _row, _mat, _bia, _mat, _mat, _bia, _mat, _bia,
                  _mat, _mat],
        out_specs=[_row, _row, _col],
        out_shape=[jax.ShapeDtypeStruct((N, DIM), F32),
                   jax.ShapeDtypeStruct((N, DIM), F32),
                   jax.ShapeDtypeStruct((N, 1), F32)],
    )(s, cnt, h, wtk('conv3_Wl'), b2k('conv3_bl'), wtk('conv3_Wr'),
      wt('lin6'), b2('lin6'), wt('gmt_lin1'), b2('gmt_lin1'),
      wt('mab1_k'), wt('mab1_v'))

    skv = _segsum_kv(hk_s, hv_s, src, dst)

    den, num = pl.pallas_call(
        _pool1_body, grid=(GRID,),
        in_specs=[_seg, _row, _row, _col, _col,
                  pl.BlockSpec((NS, DIM), lambda i: (0, 0)),
                  _mat, _bia, _bia, _bia],
        out_specs=[pl.BlockSpec((NG, NH * NS), lambda i: (0, 0)),
                   pl.BlockSpec((NH, NS * NG, HD), lambda i: (0, 0, 0))],
        out_shape=[jax.ShapeDtypeStruct((NG, NH * NS), F32),
                   jax.ShapeDtypeStruct((NH, NS * NG, HD), F32)],
    )(skv, hk_s, hv_s, dinv, batch2, p['S1'].reshape(NS, DIM),
      wt('mab1_fcq'), b2('mab1_fcq'), b2('mab1_k'), b2('mab1_v'))

    full = lambda shp: pl.BlockSpec(shp, lambda: tuple(0 for _ in shp))
    args = [num, den, p['S1'].reshape(NS, DIM), p['S3'].reshape(1, DIM),
            wt('mab1_fcq'), b2('mab1_fcq'), wt('mab1_fco'), b2('mab1_fco'),
            wt('mab2_fcq'), b2('mab2_fcq'), wt('mab2_k'), b2('mab2_k'),
            wt('mab2_v'), b2('mab2_v'), wt('mab2_fco'), b2('mab2_fco'),
            wt('mab3_fcq'), b2('mab3_fcq'), wt('mab3_k'), b2('mab3_k'),
            wt('mab3_v'), b2('mab3_v'), wt('mab3_fco'), b2('mab3_fco'),
            p['gmt_lin2_W'].T, p['gmt_lin2_b'].reshape(1, 1)]
    res = pl.pallas_call(
        _pooltail_body,
        in_specs=[full(a.shape) for a in args],
        out_specs=full((NG, 1)),
        out_shape=jax.ShapeDtypeStruct((NG, 1), F32),
    )(*args)
    return res.reshape(-1)


# speedup vs baseline: 1.4357x; 1.4357x over previous
import functools

import jax
import jax.numpy as jnp
from jax import lax
from jax.experimental import pallas as pl
from jax.experimental.pallas import tpu as pltpu
from jax.experimental.pallas import tpu_sc as plsc

N = 10000
E = 320000
DIM = 128
NG = 64
NH = 2
HD = DIM // NH
NS = 25
R = 1000
GRID = N // R
SCALE = 1.0 / (DIM ** 0.5)
F32 = jnp.float32


def _relu(v):
    return jnp.maximum(v, 0.0)


def _dot(a, b):
    return lax.dot_general(a, b, (((1,), (0,)), ((), ())),
                           preferred_element_type=F32)


def _dot3(a, w):
    return lax.dot_general(a, w, (((2,), (0,)), ((), ())),
                           preferred_element_type=F32)


_row = pl.BlockSpec((R, DIM), lambda i: (i, 0))
_row2 = pl.BlockSpec((R, 2 * DIM), lambda i: (i, 0))
_seg = pl.BlockSpec((2, R, DIM), lambda i: (0, i, 0))
_col = pl.BlockSpec((R, 1), lambda i: (i, 0))
_mat = pl.BlockSpec((DIM, DIM), lambda i: (0, 0))
_bia = pl.BlockSpec((1, DIM), lambda i: (0, 0))


def _mlp3_body(x, w1, b1, w2, b2, w3, b3, o):
    h = _relu(_dot(x[...], w1[...]) + b1[...])
    h = _relu(_dot(h, w2[...]) + b2[...])
    o[...] = _relu(_dot(h, w3[...]) + b3[...])


def _cntred_body(cnt, o):
    o[...] = cnt[0, :, 0:1] + cnt[1, :, 0:1]


def _conv_lin_body(s, cnt, h, wl, bl, wr, w4, b4, o):
    agg = (s[0] + s[1]) / jnp.maximum(cnt[...], 1.0)
    hh = _relu(_dot(agg, wl[...]) + bl[...] + _dot(h[...], wr[...]))
    o[...] = _relu(_dot(hh, w4[...]) + b4[...])


def _conv_gcnprep_body(s, cnt, h, wl, bl, wr, w6, b6, wg, bg, wk, wv,
                       hk_o, hv_o, dinv_o):
    agg = (s[0] + s[1]) / jnp.maximum(cnt[...], 1.0)
    hh = _relu(_dot(agg, wl[...]) + bl[...] + _dot(h[...], wr[...]))
    h6 = _relu(_dot(hh, w6[...]) + b6[...])
    xg = _dot(h6, wg[...]) + bg[...]
    dinv = lax.rsqrt(cnt[...] + 1.0)
    hk_o[...] = _dot(xg, wk[...]) * dinv
    hv_o[...] = _dot(xg, wv[...]) * dinv
    dinv_o[...] = dinv


def _pool1_body(skv, hk, hv, dinv, batch, s1, wq, bq, bk, bv, den_o, num_o):
    i = pl.program_id(0)

    @pl.when(i == 0)
    def _():
        den_o[...] = jnp.zeros_like(den_o)
        num_o[...] = jnp.zeros_like(num_o)

    di = dinv[...]
    K = di * (skv[0] + hk[...]) + bk[...]
    V = di * (skv[1] + hv[...]) + bv[...]
    Qp = _dot(s1[...], wq[...]) + bq[...]
    onehot = (batch[...] == lax.broadcasted_iota(jnp.int32, (R, NG), 1)
              ).astype(F32)
    for h in range(NH):
        sl = slice(h * HD, (h + 1) * HD)
        Kh, Vh, Qh = K[:, sl], V[:, sl], Qp[:, sl]
        S = lax.dot_general(Kh, Qh, (((1,), (1,)), ((), ())),
                            preferred_element_type=F32) * SCALE
        w = jnp.exp(S)
        den_o[:, h * NS:(h + 1) * NS] += lax.dot_general(
            onehot, w, (((0,), (0,)), ((), ())), preferred_element_type=F32)
        wb = (w[:, :, None] * onehot[:, None, :]).reshape(R, NS * NG)
        num_o[h] += lax.dot_general(
            wb, Vh, (((0,), (0,)), ((), ())), preferred_element_type=F32)


def _pooltail_body(num, den, s1, s3,
                   wq1, bq1, wo1, bo1,
                   wq2, bq2, wk2, bk2, wv2, bv2, wo2, bo2,
                   wq3, bq3, wk3, bk3, wv3, bv3, wo3, bo3,
                   wg2, bg2, out):
    Qp1 = _dot(s1[...], wq1[...]) + bq1[...]
    d = den[...]
    rows = []
    for q in range(NS):
        d0 = d[:, q:q + 1]
        d1 = d[:, NS + q:NS + q + 1]
        n0 = num[0, q * NG:(q + 1) * NG, :]
        n1 = num[1, q * NG:(q + 1) * NG, :]
        v0 = jnp.where(d0 > 0, n0 / jnp.maximum(d0, 1e-30), 0.0)
        v1 = jnp.where(d1 > 0, n1 / jnp.maximum(d1, 1e-30), 0.0)
        o = jnp.concatenate([v0, v1], axis=1) + Qp1[q:q + 1, :]
        rows.append(o + _relu(_dot(o, wo1[...]) + bo1[...]))
    bx = jnp.stack(rows, axis=1)

    def lin3(a, w, b):
        return _dot3(a, w[...]) + b[...]

    def sab(bx2, Qf, wk, bk_, wv, bv_, wo, bo_):
        K2 = lin3(bx2, wk, bk_)
        V2 = lin3(bx2, wv, bv_)
        outs = []
        for h in range(NH):
            sl = slice(h * HD, (h + 1) * HD)
            sc = lax.dot_general(Qf[..., sl], K2[..., sl],
                                 (((2,), (2,)), ((0,), (0,))),
                                 preferred_element_type=F32) * SCALE
            m = jnp.max(sc, axis=-1, keepdims=True)
            e = jnp.exp(sc - m)
            A = e / jnp.sum(e, axis=-1, keepdims=True)
            outs.append(Qf[..., sl] + lax.dot_general(
                A, V2[..., sl], (((2,), (1,)), ((0,), (0,))),
                preferred_element_type=F32))
        o2 = jnp.concatenate(outs, axis=-1)
        return o2 + _relu(lin3(o2, wo, bo_))

    bx = sab(bx, lin3(bx, wq2, bq2), wk2, bk2, wv2, bv2, wo2, bo2)

    Qp3 = _dot(s3[...], wq3[...]) + bq3[...]
    K3 = lin3(bx, wk3, bk3)
    V3 = lin3(bx, wv3, bv3)
    outs = []
    for h in range(NH):
        sl = slice(h * HD, (h + 1) * HD)
        sc = jnp.sum(K3[..., sl] * Qp3[0:1, None, sl], axis=-1) * SCALE
        m = jnp.max(sc, axis=-1, keepdims=True)
        e = jnp.exp(sc - m)
        A = e / jnp.sum(e, axis=-1, keepdims=True)
        outs.append(Qp3[0:1, sl] + jnp.sum(A[..., None] * V3[..., sl], axis=1))
    o3 = jnp.concatenate(outs, axis=-1)
    bx3 = o3 + _relu(_dot(o3, wo3[...]) + bo3[...])
    out[...] = _dot(bx3, wg2[...]) + bg2[...]


SC_NC = 2
SC_NS = 16
NW = SC_NC * SC_NS
EW = E // NW
CH = 80
E_PAD = E
EWP = E_PAD // NW
NCH = EWP // CH
N_PAD = 10240
ZR = N_PAD // SC_NS
WB = 80
NWB = ZR // WB
_MESH = plsc.VectorSubcoreMesh(core_axis_name="c", subcore_axis_name="s",
                               num_cores=SC_NC, num_subcores=SC_NS)


def _zero_acc(zrow, buf, acc, s):
    pltpu.sync_copy(zrow, buf)
    for j in range(NWB):
        pltpu.sync_copy(buf, acc.at[pl.ds(s * ZR + j * WB, WB)])


def _writeback(acc, buf, out, c, s):
    for j in range(NWB):
        off = s * ZR + j * WB
        pltpu.sync_copy(acc.at[pl.ds(off, WB)], buf)
        pltpu.sync_copy(buf, out.at[pl.ds(c * N_PAD + off, WB)])


def _make_segsum():
    scratch = [
        pltpu.VMEM((CH,), jnp.int32),
        pltpu.VMEM((CH,), jnp.int32),
        pltpu.VMEM((CH,), jnp.int32),
        pltpu.VMEM((CH,), jnp.int32),
        pltpu.VMEM((CH, DIM), F32),
        pltpu.VMEM((CH, DIM), F32),
        pltpu.VMEM_SHARED((N_PAD, DIM), F32),
        pltpu.SemaphoreType.DMA,
        pltpu.SemaphoreType.DMA,
        pltpu.SemaphoreType.DMA,
        pltpu.SemaphoreType.DMA,
    ]

    def body(tab, srcr, dstr, zrow, out,
             sa_i, da_i, sb_i, db_i, ra, rb, acc, ga, gb, ia, ib):
        c = lax.axis_index("c")
        s = lax.axis_index("s")
        wid = s * SC_NC + c
        base = wid * EWP
        wbuf = ra.at[pl.ds(0, WB)]
        _zero_acc(zrow, wbuf, acc, s)
        plsc.subcore_barrier()

        def load_idx(k, si, di, sem):
            pltpu.async_copy(srcr.at[pl.ds(base + k * CH, CH)], si, sem)
            pltpu.async_copy(dstr.at[pl.ds(base + k * CH, CH)], di, sem)

        def wait_idx(si, di, sem):
            pltpu.make_async_copy(srcr.at[pl.ds(0, CH)], si, sem).wait()
            pltpu.make_async_copy(dstr.at[pl.ds(0, CH)], di, sem).wait()

        load_idx(0, sa_i, da_i, ia)
        wait_idx(sa_i, da_i, ia)
        pltpu.async_copy(tab.at[sa_i], ra, ga)
        load_idx(1, sb_i, db_i, ib)

        def step(i, carry):
            pltpu.make_async_copy(tab.at[sa_i], ra, ga).wait()
            wait_idx(sb_i, db_i, ib)
            pltpu.async_copy(tab.at[sb_i], rb, gb)
            pltpu.sync_copy(ra, acc.at[da_i], add=True)

            @pl.when(2 * i + 2 < NCH)
            def _():
                load_idx(2 * i + 2, sa_i, da_i, ia)

            pltpu.make_async_copy(tab.at[sb_i], rb, gb).wait()
            pltpu.sync_copy(rb, acc.at[db_i], add=True)

            @pl.when(2 * i + 2 < NCH)
            def _():
                wait_idx(sa_i, da_i, ia)
                pltpu.async_copy(tab.at[sa_i], ra, ga)

            @pl.when(2 * i + 3 < NCH)
            def _():
                load_idx(2 * i + 3, sb_i, db_i, ib)

            return carry

        lax.fori_loop(0, NCH // 2, step, 0)
        if NCH % 2:
            pltpu.make_async_copy(tab.at[sa_i], ra, ga).wait()
            pltpu.sync_copy(ra, acc.at[da_i], add=True)
        plsc.subcore_barrier()
        _writeback(acc, ra.at[pl.ds(0, WB)], out, c, s)

    return pl.kernel(body,
                     out_type=jax.ShapeDtypeStruct((2 * N_PAD, DIM), F32),
                     mesh=_MESH, scratch_types=scratch)


def _make_segsum_kv():
    EW2 = E_PAD // SC_NS
    NCH2 = EW2 // CH
    scratch = [
        pltpu.VMEM((CH,), jnp.int32),
        pltpu.VMEM((CH,), jnp.int32),
        pltpu.VMEM((CH,), jnp.int32),
        pltpu.VMEM((CH,), jnp.int32),
        pltpu.VMEM((CH, DIM), F32),
        pltpu.VMEM((CH, DIM), F32),
        pltpu.VMEM_SHARED((N_PAD, DIM), F32),
        pltpu.SemaphoreType.DMA,
        pltpu.SemaphoreType.DMA,
        pltpu.SemaphoreType.DMA,
        pltpu.SemaphoreType.DMA,
    ]

    def body(tabk, tabv, srcr, dstr, zrow, out,
             sa_i, da_i, sb_i, db_i, ra, rb, acc, ga, gb, ia, ib):
        c = lax.axis_index("c")
        s = lax.axis_index("s")
        base = s * EW2
        wbuf = ra.at[pl.ds(0, WB)]
        _zero_acc(zrow, wbuf, acc, s)
        plsc.subcore_barrier()

        def load_idx(k, si, di, sem):
            pltpu.async_copy(srcr.at[pl.ds(base + k * CH, CH)], si, sem)
            pltpu.async_copy(dstr.at[pl.ds(base + k * CH, CH)], di, sem)

        def wait_idx(si, di, sem):
            pltpu.make_async_copy(srcr.at[pl.ds(0, CH)], si, sem).wait()
            pltpu.make_async_copy(dstr.at[pl.ds(0, CH)], di, sem).wait()

        def pipeline(tab):
            load_idx(0, sa_i, da_i, ia)
            wait_idx(sa_i, da_i, ia)
            pltpu.async_copy(tab.at[sa_i], ra, ga)
            load_idx(1, sb_i, db_i, ib)

            def step(i, carry):
                pltpu.make_async_copy(tab.at[sa_i], ra, ga).wait()
                wait_idx(sb_i, db_i, ib)
                pltpu.async_copy(tab.at[sb_i], rb, gb)
                pltpu.sync_copy(ra, acc.at[da_i], add=True)

                @pl.when(2 * i + 2 < NCH2)
                def _():
                    load_idx(2 * i + 2, sa_i, da_i, ia)

                pltpu.make_async_copy(tab.at[sb_i], rb, gb).wait()
                pltpu.sync_copy(rb, acc.at[db_i], add=True)

                @pl.when(2 * i + 2 < NCH2)
                def _():
                    wait_idx(sa_i, da_i, ia)
                    pltpu.async_copy(tab.at[sa_i], ra, ga)

                @pl.when(2 * i + 3 < NCH2)
                def _():
                    load_idx(2 * i + 3, sb_i, db_i, ib)

                return carry

            lax.fori_loop(0, NCH2 // 2, step, 0)

        @pl.when(c == 0)
        def _():
            pipeline(tabk)

        @pl.when(c == 1)
        def _():
            pipeline(tabv)

        plsc.subcore_barrier()
        _writeback(acc, ra.at[pl.ds(0, WB)], out, c, s)

    return pl.kernel(body,
                     out_type=jax.ShapeDtypeStruct((2 * N_PAD, DIM), F32),
                     mesh=_MESH, scratch_types=scratch)


def _segsum_kv(tabk, tabv, src, dst):
    zrow = jnp.zeros((WB, DIM), F32)
    srcp, dstp = _pad_edges(src, dst)
    s = _make_segsum_kv()(tabk, tabv, srcp, dstp, zrow)
    return s.reshape(2, N_PAD, DIM)


def _make_counts():
    scratch = [
        pltpu.VMEM((NCH, CH), jnp.int32),
        pltpu.VMEM((CH, DIM), F32),
        pltpu.VMEM_SHARED((N_PAD, DIM), F32),
    ]

    def body(dst3, zrow, ones, out, didx, ob, acc):
        c = lax.axis_index("c")
        s = lax.axis_index("s")
        wid = s * SC_NC + c
        wbuf = ob.at[pl.ds(0, WB)]
        _zero_acc(zrow, wbuf, acc, s)
        pltpu.sync_copy(dst3.at[wid], didx)
        pltpu.sync_copy(ones, ob)
        plsc.subcore_barrier()

        def step(i, carry):
            pltpu.sync_copy(ob, acc.at[didx.at[i]], add=True)
            return carry

        lax.fori_loop(0, NCH, step, 0)
        plsc.subcore_barrier()
        _writeback(acc, wbuf, out, c, s)

    return pl.kernel(body,
                     out_type=jax.ShapeDtypeStruct((2 * N_PAD, DIM), F32),
                     mesh=_MESH, scratch_types=scratch)


def _pad_edges(src, dst):
    return src, dst


def _segsum(tab, src, dst):
    zrow = jnp.zeros((WB, DIM), F32)
    srcp, dstp = _pad_edges(src, dst)
    s = _make_segsum()(tab, srcp, dstp, zrow)
    return s.reshape(2, N_PAD, DIM)


def _counts(dst):
    zrow = jnp.zeros((WB, DIM), F32)
    ones = jnp.ones((CH, DIM), F32)
    dst3 = dst.reshape(NW, NCH, CH)
    c = _make_counts()(dst3, zrow, ones)
    return c.reshape(2, N_PAD, DIM)


def kernel(x, params, edge_index, batch):
    p = params
    src, dst = edge_index[0], edge_index[1]

    def wt(nm):
        return p[nm + '_W'].T

    def wtk(key):
        return p[key].T

    def b2(nm):
        return p[nm + '_b'].reshape(1, DIM)

    def b2k(key):
        return p[key].reshape(1, DIM)

    batch2 = batch.reshape(N, 1)
    cnt2 = _counts(dst)
    cnt = pl.pallas_call(
        _cntred_body, grid=(GRID,),
        in_specs=[_seg],
        out_specs=_col,
        out_shape=jax.ShapeDtypeStruct((N, 1), F32),
    )(cnt2)
    _cnt = _col

    h3 = pl.pallas_call(
        _mlp3_body, grid=(GRID,),
        in_specs=[_row, _mat, _bia, _mat, _bia, _mat, _bia],
        out_specs=_row,
        out_shape=jax.ShapeDtypeStruct((N, DIM), F32),
    )(x, wt('lin1'), b2('lin1'), wt('lin2'), b2('lin2'), wt('lin3'), b2('lin3'))

    h = h3
    for conv, nxt in (('conv1', 'lin4'), ('conv2', 'lin5')):
        s = _segsum(h, src, dst)
        h = pl.pallas_call(
            _conv_lin_body, grid=(GRID,),
            in_specs=[_seg, _cnt, _row, _mat, _bia, _mat, _mat, _bia],
            out_specs=_row,
            out_shape=jax.ShapeDtypeStruct((N, DIM), F32),
        )(s, cnt, h, wtk(conv + '_Wl'), b2k(conv + '_bl'), wtk(conv + '_Wr'),
          wt(nxt), b2(nxt))

    s = _segsum(h, src, dst)
    hk_s, hv_s, dinv = pl.pallas_call(
        _conv_gcnprep_body, grid=(GRID,),
        in_specs=[_seg, _cnt, _row, _mat, _bia, _mat, _mat, _bia, _mat, _bia,
                  _mat, _mat],
        out_specs=[_row, _row, _col],
        out_shape=[jax.ShapeDtypeStruct((N, DIM), F32),
                   jax.ShapeDtypeStruct((N, DIM), F32),
                   jax.ShapeDtypeStruct((N, 1), F32)],
    )(s, cnt, h, wtk('conv3_Wl'), b2k('conv3_bl'), wtk('conv3_Wr'),
      wt('lin6'), b2('lin6'), wt('gmt_lin1'), b2('gmt_lin1'),
      wt('mab1_k'), wt('mab1_v'))

    skv = _segsum_kv(hk_s, hv_s, src, dst)

    den, num = pl.pallas_call(
        _pool1_body, grid=(GRID,),
        in_specs=[_seg, _row, _row, _col, _col,
                  pl.BlockSpec((NS, DIM), lambda i: (0, 0)),
                  _mat, _bia, _bia, _bia],
        out_specs=[pl.BlockSpec((NG, NH * NS), lambda i: (0, 0)),
                   pl.BlockSpec((NH, NS * NG, HD), lambda i: (0, 0, 0))],
        out_shape=[jax.ShapeDtypeStruct((NG, NH * NS), F32),
                   jax.ShapeDtypeStruct((NH, NS * NG, HD), F32)],
    )(skv, hk_s, hv_s, dinv, batch2, p['S1'].reshape(NS, DIM),
      wt('mab1_fcq'), b2('mab1_fcq'), b2('mab1_k'), b2('mab1_v'))

    full = lambda shp: pl.BlockSpec(shp, lambda: tuple(0 for _ in shp))
    args = [num, den, p['S1'].reshape(NS, DIM), p['S3'].reshape(1, DIM),
            wt('mab1_fcq'), b2('mab1_fcq'), wt('mab1_fco'), b2('mab1_fco'),
            wt('mab2_fcq'), b2('mab2_fcq'), wt('mab2_k'), b2('mab2_k'),
            wt('mab2_v'), b2('mab2_v'), wt('mab2_fco'), b2('mab2_fco'),
            wt('mab3_fcq'), b2('mab3_fcq'), wt('mab3_k'), b2('mab3_k'),
            wt('mab3_v'), b2('mab3_v'), wt('mab3_fco'), b2('mab3_fco'),
            p['gmt_lin2_W'].T, p['gmt_lin2_b'].reshape(1, 1)]
    res = pl.pallas_call(
        _pooltail_body,
        in_specs=[full(a.shape) for a in args],
        out_specs=full((NG, 1)),
        out_shape=jax.ShapeDtypeStruct((NG, 1), F32),
    )(*args)
    return res.reshape(-1)
